# Initial kernel scaffold; baseline (speedup 1.0000x reference)
import jax
import jax.numpy as jnp
from jax.experimental import pallas as pl


def kernel(cl, re, anc):
    raise NotImplementedError("write your pallas kernel here")


# speedup vs baseline: 533.3639x; 533.3639x over previous
import jax
import jax.numpy as jnp
from jax.experimental import pallas as pl

N = 5000
ROWS = 40
LANES = 128
NPAD = ROWS * LANES
BB_NUM = 100
THR = 0.5
CLIP_MAX = 511.0


def _nms_kernel(c0_ref, c1_ref, r0_ref, r1_ref, r2_ref, r3_ref,
                a0_ref, a1_ref, a2_ref, a3_ref, out_ref):
    c0 = c0_ref[0]
    c1 = c1_ref[0]
    e0 = jnp.exp(c0)
    e1 = jnp.exp(c1)
    ff = e0 / (e0 + e1)
    xmin = jnp.maximum(a0_ref[...] - r0_ref[0], 0.0)
    ymin = jnp.maximum(a1_ref[...] - r1_ref[0], 0.0)
    xmax = jnp.minimum(a2_ref[...] + r2_ref[0], CLIP_MAX)
    ymax = jnp.minimum(a3_ref[...] + r3_ref[0], CLIP_MAX)
    areas = jnp.maximum(xmax - xmin, 0.0) * jnp.maximum(ymax - ymin, 0.0)
    row = jax.lax.broadcasted_iota(jnp.int32, (ROWS, LANES), 0)
    lane = jax.lax.broadcasted_iota(jnp.int32, (ROWS, LANES), 1)
    flat = row * LANES + lane
    supp0 = jnp.where(flat >= N, 1.0, 0.0)
    olane = jax.lax.broadcasted_iota(jnp.int32, (1, LANES), 1)
    zrow = jnp.zeros((1, LANES), jnp.float32)

    def body(t, carry):
        supp, ob0, ob1, ob2, ob3, osc, l0, l1, l2, l3, ls = carry
        msc = jnp.where(supp > 0.0, -1.0, ff)
        m = jnp.max(msc)
        valid = m >= 0.0
        sel = jnp.min(jnp.where(msc == m, flat, NPAD))
        hit = flat == sel
        bx0 = jnp.sum(jnp.where(hit, xmin, 0.0))
        bx1 = jnp.sum(jnp.where(hit, ymin, 0.0))
        bx2 = jnp.sum(jnp.where(hit, xmax, 0.0))
        bx3 = jnp.sum(jnp.where(hit, ymax, 0.0))
        v0 = jnp.where(valid, bx0, l0)
        v1 = jnp.where(valid, bx1, l1)
        v2 = jnp.where(valid, bx2, l2)
        v3 = jnp.where(valid, bx3, l3)
        vs = jnp.where(valid, m, ls)
        tm = olane == t
        ob0 = jnp.where(tm, v0, ob0)
        ob1 = jnp.where(tm, v1, ob1)
        ob2 = jnp.where(tm, v2, ob2)
        ob3 = jnp.where(tm, v3, ob3)
        osc = jnp.where(tm, vs, osc)
        selarea = jnp.maximum(bx2 - bx0, 0.0) * jnp.maximum(bx3 - bx1, 0.0)
        xx1 = jnp.maximum(xmin, bx0)
        yy1 = jnp.maximum(ymin, bx1)
        xx2 = jnp.minimum(xmax, bx2)
        yy2 = jnp.minimum(ymax, bx3)
        inter = jnp.maximum(xx2 - xx1, 0.0) * jnp.maximum(yy2 - yy1, 0.0)
        iou = inter / (areas + selarea - inter + 1e-9)
        nsupp = jnp.maximum(supp, jnp.where((iou > THR) | hit, 1.0, 0.0))
        supp = jnp.where(valid, nsupp, supp)
        return (supp, ob0, ob1, ob2, ob3, osc, v0, v1, v2, v3, vs)

    init = (supp0, zrow, zrow, zrow, zrow, zrow, 0.0, 0.0, 0.0, 0.0, 0.0)
    res = jax.lax.fori_loop(0, BB_NUM, body, init)
    out = jnp.concatenate([res[1], res[2], res[3], res[4], res[5],
                           zrow, zrow, zrow], axis=0)
    out_ref[0] = out


def kernel(cl, re, anc):
    B = cl.shape[0]
    pad = NPAD - N

    def prep(x):
        return jnp.pad(x, ((0, 0), (0, pad))).reshape(B, ROWS, LANES)

    def prepa(x):
        return jnp.pad(x, (0, pad)).reshape(ROWS, LANES)

    c0 = prep(cl[..., 0])
    c1 = prep(cl[..., 1])
    r0 = prep(re[..., 0])
    r1 = prep(re[..., 1])
    r2 = prep(re[..., 2])
    r3 = prep(re[..., 3])
    a0 = prepa(anc[0, :, 0])
    a1 = prepa(anc[0, :, 1])
    a2 = prepa(anc[0, :, 2])
    a3 = prepa(anc[0, :, 3])

    bspec = pl.BlockSpec((1, ROWS, LANES), lambda b: (b, 0, 0))
    aspec = pl.BlockSpec((ROWS, LANES), lambda b: (0, 0))
    out = pl.pallas_call(
        _nms_kernel,
        grid=(B,),
        in_specs=[bspec] * 6 + [aspec] * 4,
        out_specs=pl.BlockSpec((1, 8, LANES), lambda b: (b, 0, 0)),
        out_shape=jax.ShapeDtypeStruct((B, 8, LANES), jnp.float32),
    )(c0, c1, r0, r1, r2, r3, a0, a1, a2, a3)

    bb = jnp.stack([out[:, 0, :BB_NUM], out[:, 1, :BB_NUM],
                    out[:, 2, :BB_NUM], out[:, 3, :BB_NUM]], axis=-1)
    ffo = out[:, 4, :BB_NUM]
    return bb, ffo


# speedup vs baseline: 1861.8762x; 3.4908x over previous
import jax
import jax.numpy as jnp
from jax.experimental import pallas as pl

N = 5000
ROWS = 40
LANES = 128
NPAD = ROWS * LANES
BB_NUM = 100
THR = 0.5
CLIP_MAX = 511.0
B = 4


def _nms_kernel(c0_ref, c1_ref, r0_ref, r1_ref, r2_ref, r3_ref,
                a0_ref, a1_ref, a2_ref, a3_ref, out_ref):
    c0 = c0_ref[...]
    c1 = c1_ref[...]
    e0 = jnp.exp(c0)
    e1 = jnp.exp(c1)
    ff = e0 / (e0 + e1)
    anc0 = a0_ref[...][None]
    anc1 = a1_ref[...][None]
    anc2 = a2_ref[...][None]
    anc3 = a3_ref[...][None]
    xmin = jnp.maximum(anc0 - r0_ref[...], 0.0)
    ymin = jnp.maximum(anc1 - r1_ref[...], 0.0)
    xmax = jnp.minimum(anc2 + r2_ref[...], CLIP_MAX)
    ymax = jnp.minimum(anc3 + r3_ref[...], CLIP_MAX)
    areas = jnp.maximum(xmax - xmin, 0.0) * jnp.maximum(ymax - ymin, 0.0)
    row = jax.lax.broadcasted_iota(jnp.int32, (B, ROWS, LANES), 1)
    lane = jax.lax.broadcasted_iota(jnp.int32, (B, ROWS, LANES), 2)
    flat = row * LANES + lane
    supp0 = jnp.where(flat >= N, 1.0, 0.0)
    olane = jax.lax.broadcasted_iota(jnp.int32, (B, 1, LANES), 2)
    zout = jnp.zeros((B, 1, LANES), jnp.float32)
    zsc = jnp.zeros((B, 1, 1), jnp.float32)

    def body(t, carry):
        supp, ob0, ob1, ob2, ob3, osc, l0, l1, l2, l3, ls = carry
        msc = jnp.where(supp > 0.0, -1.0, ff)
        m = jnp.max(msc, axis=(1, 2), keepdims=True)
        valid = m >= 0.0
        sel = jnp.min(jnp.where(msc == m, flat, NPAD), axis=(1, 2),
                      keepdims=True)
        hit = flat == sel
        bx0 = jnp.sum(jnp.where(hit, xmin, 0.0), axis=(1, 2), keepdims=True)
        bx1 = jnp.sum(jnp.where(hit, ymin, 0.0), axis=(1, 2), keepdims=True)
        bx2 = jnp.sum(jnp.where(hit, xmax, 0.0), axis=(1, 2), keepdims=True)
        bx3 = jnp.sum(jnp.where(hit, ymax, 0.0), axis=(1, 2), keepdims=True)
        v0 = jnp.where(valid, bx0, l0)
        v1 = jnp.where(valid, bx1, l1)
        v2 = jnp.where(valid, bx2, l2)
        v3 = jnp.where(valid, bx3, l3)
        vs = jnp.where(valid, m, ls)
        tm = olane == t
        ob0 = jnp.where(tm, v0, ob0)
        ob1 = jnp.where(tm, v1, ob1)
        ob2 = jnp.where(tm, v2, ob2)
        ob3 = jnp.where(tm, v3, ob3)
        osc = jnp.where(tm, vs, osc)
        selarea = jnp.maximum(bx2 - bx0, 0.0) * jnp.maximum(bx3 - bx1, 0.0)
        xx1 = jnp.maximum(xmin, bx0)
        yy1 = jnp.maximum(ymin, bx1)
        xx2 = jnp.minimum(xmax, bx2)
        yy2 = jnp.minimum(ymax, bx3)
        inter = jnp.maximum(xx2 - xx1, 0.0) * jnp.maximum(yy2 - yy1, 0.0)
        iou = inter / (areas + selarea - inter + 1e-9)
        nsupp = jnp.maximum(supp, jnp.where((iou > THR) | hit, 1.0, 0.0))
        supp = jnp.where(valid, nsupp, supp)
        return (supp, ob0, ob1, ob2, ob3, osc, v0, v1, v2, v3, vs)

    init = (supp0, zout, zout, zout, zout, zout, zsc, zsc, zsc, zsc, zsc)
    res = jax.lax.fori_loop(0, BB_NUM, body, init)
    zrow = jnp.zeros((B, 1, LANES), jnp.float32)
    out = jnp.concatenate([res[1], res[2], res[3], res[4], res[5],
                           zrow, zrow, zrow], axis=1)
    out_ref[...] = out


def kernel(cl, re, anc):
    pad = NPAD - N

    def prep(x):
        return jnp.pad(x, ((0, 0), (0, pad))).reshape(B, ROWS, LANES)

    def prepa(x):
        return jnp.pad(x, (0, pad)).reshape(ROWS, LANES)

    c0 = prep(cl[..., 0])
    c1 = prep(cl[..., 1])
    r0 = prep(re[..., 0])
    r1 = prep(re[..., 1])
    r2 = prep(re[..., 2])
    r3 = prep(re[..., 3])
    a0 = prepa(anc[0, :, 0])
    a1 = prepa(anc[0, :, 1])
    a2 = prepa(anc[0, :, 2])
    a3 = prepa(anc[0, :, 3])

    out = pl.pallas_call(
        _nms_kernel,
        out_shape=jax.ShapeDtypeStruct((B, 8, LANES), jnp.float32),
    )(c0, c1, r0, r1, r2, r3, a0, a1, a2, a3)

    bb = jnp.stack([out[:, 0, :BB_NUM], out[:, 1, :BB_NUM],
                    out[:, 2, :BB_NUM], out[:, 3, :BB_NUM]], axis=-1)
    ffo = out[:, 4, :BB_NUM]
    return bb, ffo
